# baseline (speedup 1.0000x reference)
import functools

import jax
import jax.numpy as jnp
from jax import lax
from jax.experimental import pallas as pl
from jax.experimental.pallas import tpu as pltpu
from jax.experimental.pallas import tpu_sc as plsc

_EMBED_DIM = 32
_DEFINED = (90.0, 109.5, 120.0, 180.0, 0.0)
_TOL = 5.0
_L = 16

_NC, _NS = 2, 16
_NW = _NC * _NS
_CHUNK = 512


def _bucket_index(a):
    idx = jnp.zeros((_L,), jnp.int32)
    tol = jnp.full((_L,), _TOL, jnp.float32)
    for j in range(len(_DEFINED) - 1, -1, -1):
        m = jnp.abs(a - jnp.full((_L,), _DEFINED[j], jnp.float32)) <= tol
        idx = jnp.where(m, jnp.full((_L,), j, jnp.int32), idx)
    return idx


def _make_sc_kernel(n):
    per_w = n // _NW
    n_chunks = per_w // _CHUNK
    mesh = plsc.VectorSubcoreMesh(core_axis_name="c", subcore_axis_name="s")

    @functools.partial(
        pl.kernel,
        out_type=jax.ShapeDtypeStruct((n * _EMBED_DIM,), jnp.float32),
        mesh=mesh,
        compiler_params=pltpu.CompilerParams(
            use_tc_tiling_on_sc=False, needs_layout_passes=False),
        scratch_types=[
            pltpu.VMEM((_CHUNK,), jnp.float32),
            pltpu.VMEM((_CHUNK,), jnp.float32),
            pltpu.VMEM((_CHUNK * _EMBED_DIM,), jnp.float32),
            pltpu.VMEM((_CHUNK * _EMBED_DIM,), jnp.float32),
            pltpu.VMEM((_CHUNK * _EMBED_DIM,), jnp.float32),
            pltpu.VMEM((_CHUNK * _EMBED_DIM,), jnp.float32),
            pltpu.VMEM((5 * _EMBED_DIM,), jnp.float32),
            pltpu.SemaphoreType.DMA,
            pltpu.SemaphoreType.DMA,
            pltpu.SemaphoreType.DMA,
            pltpu.SemaphoreType.DMA,
            pltpu.SemaphoreType.DMA,
            pltpu.SemaphoreType.DMA,
        ],
    )
    def sc_kernel(ang_hbm, emb_hbm, out_hbm,
                  ang0, ang1, rows0, rows1, rows2, rows3, tbl_v,
                  in_sem0, in_sem1, out_sem0, out_sem1, out_sem2, out_sem3):
        wid = lax.axis_index("s") * _NC + lax.axis_index("c")
        wbase = wid * per_w
        angs = (ang0, ang1)
        rows = (rows0, rows1, rows2, rows3)
        in_sems = (in_sem0, in_sem1)
        out_sems = (out_sem0, out_sem1, out_sem2, out_sem3)
        io16 = lax.iota(jnp.int32, _L)

        pltpu.sync_copy(emb_hbm, tbl_v)

        def ang_in(t, b):
            base = pl.multiple_of(wbase + t * _CHUNK, _CHUNK)
            return pltpu.async_copy(
                ang_hbm.at[pl.ds(base, _CHUNK)], angs[b], in_sems[b])

        def expand_chunk(b):
            def grp(g, c):
                a = angs[b][pl.ds(g * _L, _L)]
                idx32 = _bucket_index(a) * jnp.full((_L,), _EMBED_DIM, jnp.int32)
                e32 = (jnp.full((_L,), g * _L, jnp.int32) + io16) \
                    * jnp.full((_L,), _EMBED_DIM, jnp.int32)
                for d in range(_EMBED_DIM):
                    dd = jnp.full((_L,), d, jnp.int32)
                    vals = plsc.load_gather(tbl_v, [idx32 + dd])
                    plsc.store_scatter(rows[b], [e32 + dd], vals)
                return c
            lax.fori_loop(0, _CHUNK // _L, grp, 0)

        def rows_out(t, b):
            base = pl.multiple_of((wbase + t * _CHUNK) * _EMBED_DIM,
                                  _CHUNK * _EMBED_DIM)
            return pltpu.async_copy(
                rows[b], out_hbm.at[pl.ds(base, _CHUNK * _EMBED_DIM)],
                out_sems[b])

        def drain_in(b):
            pltpu.make_async_copy(
                ang_hbm.at[pl.ds(0, _CHUNK)], angs[b], in_sems[b]).wait()

        def drain_out(b):
            pltpu.make_async_copy(
                rows[b], out_hbm.at[pl.ds(0, _CHUNK * _EMBED_DIM)],
                out_sems[b]).wait()

        ang_in(0, 0)

        def pair(t2, c):
            for b4 in range(4):
                t = t2 * 4 + b4
                b = b4 % 2

                drain_in(b)

                @pl.when(t + 1 < n_chunks)
                def _():
                    ang_in(t + 1, 1 - b)

                @pl.when(t >= 4)
                def _():
                    drain_out(b4)


                rows_out(t, b4)
            return c

        lax.fori_loop(0, n_chunks // 4, pair, 0)
        drain_out(0)
        drain_out(1)
        drain_out(2)
        drain_out(3)

    return sc_kernel


def kernel(angles, embedding):
    n = angles.shape[0] * angles.shape[1]
    out = _make_sc_kernel(n)(angles.reshape(-1), embedding.reshape(-1))
    return out.reshape(angles.shape + (_EMBED_DIM,))


# speedup vs baseline: 8.7089x; 8.7089x over previous
import functools

import jax
import jax.numpy as jnp
from jax import lax
from jax.experimental import pallas as pl
from jax.experimental.pallas import tpu as pltpu
from jax.experimental.pallas import tpu_sc as plsc

_EMBED_DIM = 32
_DEFINED = (90.0, 109.5, 120.0, 180.0, 0.0)
_TOL = 5.0
_L = 16

_NC, _NS = 2, 16
_NW = _NC * _NS

_B = 16384
_SEQ = 200
_DT = _EMBED_DIM // 8
_BT = _B // 128
_PAIRS = _SEQ * _DT
_PPW = _PAIRS // _NW
_BT4 = 32
_ROWS_F = _BT4 * 8 * 128


def _bucket_index(a):
    idx = jnp.zeros((_L,), jnp.int32)
    tol = jnp.full((_L,), _TOL, jnp.float32)
    for j in range(len(_DEFINED) - 1, -1, -1):
        m = jnp.abs(a - jnp.full((_L,), _DEFINED[j], jnp.float32)) <= tol
        idx = jnp.where(m, jnp.full((_L,), j, jnp.int32), idx)
    return idx


def _make_sc_kernel():
    mesh = plsc.VectorSubcoreMesh(core_axis_name="c", subcore_axis_name="s")

    @functools.partial(
        pl.kernel,
        out_type=jax.ShapeDtypeStruct((_SEQ, _DT, _BT, 8, 128), jnp.float32),
        mesh=mesh,
        compiler_params=pltpu.CompilerParams(
            use_tc_tiling_on_sc=False, needs_layout_passes=False),
        scratch_types=[
            pltpu.VMEM((_B,), jnp.float32),
            pltpu.VMEM((_B,), jnp.int32),
            pltpu.VMEM((_BT4, 8, 128), jnp.float32),
            pltpu.VMEM((_BT4, 8, 128), jnp.float32),
            pltpu.VMEM((8 * _EMBED_DIM,), jnp.float32),
            pltpu.SemaphoreType.DMA,
            pltpu.SemaphoreType.DMA,
        ],
    )
    def sc_kernel(ang_t_hbm, emb_t_hbm, out_hbm,
                  ang_v, idx_v, rows0, rows1, tbl_v,
                  out_sem0, out_sem1):
        wid = lax.axis_index("s") * _NC + lax.axis_index("c")
        p0 = wid * _PPW
        rows = (rows0, rows1)
        out_sems = (out_sem0, out_sem1)

        pltpu.sync_copy(emb_t_hbm, tbl_v)

        def compute_idx(l):
            pltpu.sync_copy(ang_t_hbm.at[l], ang_v)

            def grp(g, c):
                a = ang_v[pl.ds(g * _L, _L)]
                idx_v[pl.ds(g * _L, _L)] = _bucket_index(a)
                return c

            lax.fori_loop(0, _B // _L, grp, 0)

        def drain_out(bb):
            pltpu.make_async_copy(
                rows[bb], out_hbm.at[0, 0, pl.ds(0, _BT4)], out_sems[bb]).wait()

        def pair_body(i, prev_l):
            p = p0 + i
            l = lax.shift_right_logical(p, 2)
            dt = lax.bitwise_and(p, 3)
            d5_base = dt * jnp.int32(40)

            @pl.when(l != prev_l)
            def _():
                compute_idx(l)

            for c4 in range(4):
                q = i * 4 + c4
                bb = c4 % 2

                @pl.when(q >= 2)
                def _():
                    drain_out(bb)

                base_b = c4 * (_BT4 * 128)

                def unit(u):
                    boff = lax.shift_right_logical(u, 3)
                    brg = lax.bitwise_and(u, 7)
                    bstart = base_b + boff * 128 + brg * _L
                    idxv = idx_v[pl.ds(bstart, _L)]
                    addr0 = idxv + jnp.broadcast_to(d5_base, (_L,))
                    for dr in range(8):
                        vals = plsc.load_gather(
                            tbl_v, [addr0 + jnp.full((_L,), dr * 5, jnp.int32)])
                        rows[bb][boff, dr, pl.ds(brg * _L, _L)] = vals

                plsc.parallel_loop(0, _BT4 * 8, 1, unroll=4)(unit)

                pltpu.async_copy(
                    rows[bb],
                    out_hbm.at[l, dt, pl.ds(c4 * _BT4, _BT4)],
                    out_sems[bb])
            return l

        lax.fori_loop(0, _PPW, pair_body, jnp.int32(-1))
        drain_out(0)
        drain_out(1)

    return sc_kernel


def kernel(angles, embedding):
    ang_t = angles.T
    emb_t = embedding.T.reshape(-1)
    emb_t_pad = jnp.concatenate(
        [emb_t, jnp.zeros((8 * _EMBED_DIM - 5 * _EMBED_DIM,), jnp.float32)])
    w = _make_sc_kernel()(ang_t, emb_t_pad)
    return w.transpose(2, 4, 0, 1, 3).reshape(_B, _SEQ, _EMBED_DIM)


# speedup vs baseline: 8.8351x; 1.0145x over previous
import functools

import jax
import jax.numpy as jnp
from jax import lax
from jax.experimental import pallas as pl
from jax.experimental.pallas import tpu as pltpu
from jax.experimental.pallas import tpu_sc as plsc

_EMBED_DIM = 32
_DEFINED = (90.0, 109.5, 120.0, 180.0, 0.0)
_TOL = 5.0
_L = 16

_NC, _NS = 2, 16
_NW = _NC * _NS

_B = 16384
_SEQ = 200
_DT = _EMBED_DIM // 8
_BT = _B // 128
_PAIRS = _SEQ * _DT
_PPW = _PAIRS // _NW
_BT4 = 32
_ROWS_F = _BT4 * 8 * 128


def _bucket_index(a):
    idx = jnp.zeros((_L,), jnp.int32)
    tol = jnp.full((_L,), _TOL, jnp.float32)
    for j in range(len(_DEFINED) - 1, -1, -1):
        m = jnp.abs(a - jnp.full((_L,), _DEFINED[j], jnp.float32)) <= tol
        idx = jnp.where(m, jnp.full((_L,), j, jnp.int32), idx)
    return idx


def _make_sc_kernel():
    mesh = plsc.VectorSubcoreMesh(core_axis_name="c", subcore_axis_name="s")

    @functools.partial(
        pl.kernel,
        out_type=jax.ShapeDtypeStruct((_SEQ, _DT, _BT, 8, 128), jnp.float32),
        mesh=mesh,
        compiler_params=pltpu.CompilerParams(
            use_tc_tiling_on_sc=False, needs_layout_passes=False),
        scratch_types=[
            pltpu.VMEM((_B,), jnp.float32),
            pltpu.VMEM((_B,), jnp.int32),
            pltpu.VMEM((_BT4, 8, 128), jnp.float32),
            pltpu.VMEM((_BT4, 8, 128), jnp.float32),
            pltpu.VMEM((8 * _EMBED_DIM,), jnp.float32),
            pltpu.SemaphoreType.DMA,
            pltpu.SemaphoreType.DMA,
        ],
    )
    def sc_kernel(ang_t_hbm, emb_t_hbm, out_hbm,
                  ang_v, idx_v, rows0, rows1, tbl_v,
                  out_sem0, out_sem1):
        wid = lax.axis_index("s") * _NC + lax.axis_index("c")
        p0 = wid * _PPW
        rows = (rows0, rows1)
        out_sems = (out_sem0, out_sem1)

        pltpu.sync_copy(emb_t_hbm, tbl_v)

        def compute_idx(l):
            pltpu.sync_copy(ang_t_hbm.at[l], ang_v)

            def grp(g):
                a = ang_v[pl.ds(g * _L, _L)]
                idx_v[pl.ds(g * _L, _L)] = _bucket_index(a)

            plsc.parallel_loop(0, _B // _L, 1, unroll=4)(grp)

        def drain_out(bb):
            pltpu.make_async_copy(
                rows[bb], out_hbm.at[0, 0, pl.ds(0, _BT4)], out_sems[bb]).wait()

        def pair_body(i, prev_l):
            p = p0 + i
            l = lax.shift_right_logical(p, 2)
            dt = lax.bitwise_and(p, 3)
            d5_base = dt * jnp.int32(40)

            @pl.when(l != prev_l)
            def _():
                compute_idx(l)

            for c4 in range(4):
                q = i * 4 + c4
                bb = c4 % 2

                @pl.when(q >= 2)
                def _():
                    drain_out(bb)

                base_b = c4 * (_BT4 * 128)

                def unit(u):
                    boff = lax.shift_right_logical(u, 3)
                    brg = lax.bitwise_and(u, 7)
                    bstart = base_b + boff * 128 + brg * _L
                    idxv = idx_v[pl.ds(bstart, _L)]
                    addr0 = idxv + jnp.broadcast_to(d5_base, (_L,))
                    for dr in range(8):
                        vals = plsc.load_gather(
                            tbl_v, [addr0 + jnp.full((_L,), dr * 5, jnp.int32)])
                        rows[bb][boff, dr, pl.ds(brg * _L, _L)] = vals

                plsc.parallel_loop(0, _BT4 * 8, 1, unroll=8)(unit)

                pltpu.async_copy(
                    rows[bb],
                    out_hbm.at[l, dt, pl.ds(c4 * _BT4, _BT4)],
                    out_sems[bb])
            return l

        lax.fori_loop(0, _PPW, pair_body, jnp.int32(-1))
        drain_out(0)
        drain_out(1)

    return sc_kernel


def kernel(angles, embedding):
    ang_t = angles.T
    emb_t = embedding.T.reshape(-1)
    emb_t_pad = jnp.concatenate(
        [emb_t, jnp.zeros((8 * _EMBED_DIM - 5 * _EMBED_DIM,), jnp.float32)])
    w = _make_sc_kernel()(ang_t, emb_t_pad)
    return w.transpose(2, 4, 0, 1, 3).reshape(_B, _SEQ, _EMBED_DIM)
